# Initial kernel scaffold; baseline (speedup 1.0000x reference)
import functools

import jax
import jax.numpy as jnp
from jax import lax
from jax.experimental import pallas as pl
from jax.experimental.pallas import tpu as pltpu
from jax.experimental.pallas import tpu_sc as plsc

B, T, D = 4096, 200, 128
NC, NS = 2, 16
NW = NC * NS
BPW = B // NW
LANES = 16
NFULL = T // LANES
TAIL = T - NFULL * LANES


def _sc_body(x_hbm, mask_hbm, out_hbm, mask_v, idx_v, rows_v, sem):
    wid = lax.axis_index("s") * NC + lax.axis_index("c")
    base = wid * BPW
    pltpu.sync_copy(mask_hbm.at[pl.ds(base * T, BPW * T)], mask_v)

    lane = lax.iota(jnp.int32, LANES)
    tail_keep = lane >= (LANES - TAIL)

    def row_body(r, carry):
        off = r * T
        acc = mask_v[pl.ds(off, LANES)]
        for k in range(1, NFULL):
            acc = acc + mask_v[pl.ds(off + k * LANES, LANES)]
        tail = mask_v[pl.ds(off + T - LANES, LANES)]
        acc = acc + jnp.where(tail_keep, tail, 0)
        s = jnp.sum(acc)
        row = jnp.where(s == 0, T - 1, s - 1)
        idx_v[r] = (base + r) * T + row
        return carry

    lax.fori_loop(0, BPW, row_body, 0)

    pltpu.async_copy(x_hbm.at[idx_v], rows_v, sem).wait()
    pltpu.sync_copy(rows_v, out_hbm.at[pl.ds(base, BPW)])


def kernel(x, dim, mask):
    del dim
    mesh = plsc.VectorSubcoreMesh(core_axis_name="c", subcore_axis_name="s")
    run = pl.kernel(
        _sc_body,
        out_type=jax.ShapeDtypeStruct((B, D), jnp.float32),
        mesh=mesh,
        scratch_types=[
            pltpu.VMEM((BPW * T,), jnp.int32),
            pltpu.VMEM((BPW,), jnp.int32),
            pltpu.VMEM((BPW, D), jnp.float32),
            pltpu.SemaphoreType.DMA,
        ],
    )
    out = run(x.reshape(B * T, D), mask.reshape(B * T))
    return out.reshape(B, 1, D)


# speedup vs baseline: 2.5506x; 2.5506x over previous
import functools

import jax
import jax.numpy as jnp
from jax import lax
from jax.experimental import pallas as pl
from jax.experimental.pallas import tpu as pltpu
from jax.experimental.pallas import tpu_sc as plsc

B, T, D = 4096, 200, 128
NC, NS = 2, 16
NW = NC * NS
BPW = B // NW
LANES = 16
NFULL = T // LANES
TAIL = T - NFULL * LANES


def _permute(vec, idx):
    return lax.gather(
        vec,
        idx[:, None],
        lax.GatherDimensionNumbers(
            offset_dims=(), collapsed_slice_dims=(0,), start_index_map=(0,)
        ),
        slice_sizes=(1,),
        mode=lax.GatherScatterMode.PROMISE_IN_BOUNDS,
    )


def _sc_body(x_hbm, mask_hbm, out_hbm, mask_v, idx_v, rows_v, sem):
    wid = lax.axis_index("s") * NC + lax.axis_index("c")
    base = wid * BPW
    pltpu.sync_copy(mask_hbm.at[pl.ds(base * T, BPW * T)], mask_v)

    lane = lax.iota(jnp.int32, LANES)
    tail_keep = lane >= (LANES - TAIL)
    perms = [(lane + sh) & (LANES - 1) for sh in (1, 2, 4, 8)]

    def group_body(g, carry):
        sums = jnp.zeros((LANES,), jnp.int32)
        for j in range(LANES):
            off = (g * LANES + j) * T
            acc = mask_v[pl.ds(off, LANES)]
            for k in range(1, NFULL):
                acc = acc + mask_v[pl.ds(off + k * LANES, LANES)]
            tail = mask_v[pl.ds(off + T - LANES, LANES)]
            acc = acc + jnp.where(tail_keep, tail, 0)
            for p in perms:
                acc = acc + _permute(acc, p)
            sums = jnp.where(lane == j, acc, sums)
        rows = jnp.where(sums == 0, T - 1, sums - 1)
        flat = (base + g * LANES + lane) * T + rows
        idx_v[pl.ds(g * LANES, LANES)] = flat
        return carry

    lax.fori_loop(0, BPW // LANES, group_body, 0)

    pltpu.async_copy(x_hbm.at[idx_v], rows_v, sem).wait()
    pltpu.sync_copy(rows_v, out_hbm.at[pl.ds(base, BPW)])


def kernel(x, dim, mask):
    del dim
    mesh = plsc.VectorSubcoreMesh(core_axis_name="c", subcore_axis_name="s")
    run = pl.kernel(
        _sc_body,
        out_type=jax.ShapeDtypeStruct((B, D), jnp.float32),
        mesh=mesh,
        scratch_types=[
            pltpu.VMEM((BPW * T,), jnp.int32),
            pltpu.VMEM((BPW,), jnp.int32),
            pltpu.VMEM((BPW, D), jnp.float32),
            pltpu.SemaphoreType.DMA,
        ],
    )
    out = run(x.reshape(B * T, D), mask.reshape(B * T))
    return out.reshape(B, 1, D)


# speedup vs baseline: 2.5933x; 1.0167x over previous
import functools

import jax
import jax.numpy as jnp
from jax import lax
from jax.experimental import pallas as pl
from jax.experimental.pallas import tpu as pltpu
from jax.experimental.pallas import tpu_sc as plsc

B, T, D = 4096, 200, 128
NC, NS = 2, 16
NW = NC * NS
BPW = B // NW
LANES = 16
NFULL = T // LANES
TAIL = T - NFULL * LANES


def _permute(vec, idx):
    return lax.gather(
        vec,
        idx[:, None],
        lax.GatherDimensionNumbers(
            offset_dims=(), collapsed_slice_dims=(0,), start_index_map=(0,)
        ),
        slice_sizes=(1,),
        mode=lax.GatherScatterMode.PROMISE_IN_BOUNDS,
    )


def _sc_body(x_hbm, mask_hbm, out_hbm, mask_v, idx_v, rows_v, sem):
    wid = lax.axis_index("s") * NC + lax.axis_index("c")
    base = wid * BPW
    pltpu.sync_copy(mask_hbm.at[pl.ds(base, BPW)], mask_v)

    lane = lax.iota(jnp.int32, LANES)
    tail_keep = lane >= (LANES - TAIL)
    perms = [(lane + sh) & (LANES - 1) for sh in (1, 2, 4, 8)]

    def group_body(g, carry):
        sums = jnp.zeros((LANES,), jnp.int32)
        for j in range(LANES):
            row = g * LANES + j
            acc = mask_v[row, 0, pl.ds(0, LANES)]
            for k in range(1, NFULL):
                acc = acc + mask_v[row, 0, pl.ds(k * LANES, LANES)]
            tail = mask_v[row, 0, pl.ds(T - LANES, LANES)]
            acc = acc + jnp.where(tail_keep, tail, 0)
            for p in perms:
                acc = acc + _permute(acc, p)
            sums = jnp.where(lane == j, acc, sums)
        rows = jnp.where(sums == 0, T - 1, sums - 1)
        flat = (base + g * LANES + lane) * T + rows
        idx_v[pl.ds(g * LANES, LANES)] = flat
        return carry

    lax.fori_loop(0, BPW // LANES, group_body, 0)

    pltpu.async_copy(x_hbm.at[idx_v], rows_v, sem).wait()
    pltpu.sync_copy(rows_v, out_hbm.at[pl.ds(base, BPW)])


def kernel(x, dim, mask):
    del dim
    mesh = plsc.VectorSubcoreMesh(core_axis_name="c", subcore_axis_name="s")
    run = pl.kernel(
        _sc_body,
        out_type=jax.ShapeDtypeStruct((B, D), jnp.float32),
        mesh=mesh,
        scratch_types=[
            pltpu.VMEM((BPW, 1, T), jnp.int32),
            pltpu.VMEM((BPW,), jnp.int32),
            pltpu.VMEM((BPW, D), jnp.float32),
            pltpu.SemaphoreType.DMA,
        ],
    )
    out = run(x.reshape(B * T, D), mask)
    return out.reshape(B, 1, D)
